# Initial kernel scaffold; baseline (speedup 1.0000x reference)
import jax
import jax.numpy as jnp
from jax.experimental import pallas as pl


def kernel(u_idx, i_idx, P, Q):
    raise NotImplementedError("write your pallas kernel here")


# speedup vs baseline: 1.1810x; 1.1810x over previous
import functools

import jax
import jax.numpy as jnp
from jax import lax
from jax.experimental import pallas as pl
from jax.experimental.pallas import tpu as pltpu
from jax.experimental.pallas import tpu_sc as plsc

B = 16384
F = 128
C = 128
NBUF = 2


def _dot_chunk(p_ref, q_ref, out_ref, out_base):
    lanes = lax.iota(jnp.int32, 16)
    perms = [lanes ^ s for s in (8, 4, 2, 1)]

    def hsum(v):
        for perm in perms:
            v = v + v.at[perm].get(mode="promise_in_bounds", unique_indices=True)
        return v

    def group(g, carry):
        res = jnp.zeros((16,), jnp.float32)
        for l in range(16):
            r = g * 16 + l
            acc = p_ref[r, pl.ds(0, 16)] * q_ref[r, pl.ds(0, 16)]
            for c in range(1, F // 16):
                acc = acc + p_ref[r, pl.ds(c * 16, 16)] * q_ref[r, pl.ds(c * 16, 16)]
            res = jnp.where(lanes == l, hsum(acc), res)
        out_ref[pl.ds(out_base + g * 16, 16)] = res
        return carry

    lax.fori_loop(0, C // 16, group, 0)


def kernel(u_idx, i_idx, P, Q):
    info = plsc.get_sparse_core_info()
    nc, ns = info.num_cores, info.num_subcores
    nw = nc * ns
    bpw = B // nw
    nch = bpw // C

    mesh = plsc.VectorSubcoreMesh(core_axis_name="c", subcore_axis_name="s")

    @functools.partial(
        pl.kernel,
        mesh=mesh,
        out_type=jax.ShapeDtypeStruct((B,), jnp.float32),
        scratch_types=[
            pltpu.VMEM((bpw,), jnp.int32),
            pltpu.VMEM((bpw,), jnp.int32),
            pltpu.VMEM((NBUF, C, F), jnp.float32),
            pltpu.VMEM((NBUF, C, F), jnp.float32),
            pltpu.VMEM((bpw,), jnp.float32),
            pltpu.SemaphoreType.DMA,
            pltpu.SemaphoreType.DMA,
        ],
    )
    def run(u_hbm, i_hbm, p_hbm, q_hbm, out_hbm, u_v, i_v, p_buf, q_buf,
            out_v, sem0, sem1):
        sems = [sem0, sem1]
        wid = lax.axis_index("s") * nc + lax.axis_index("c")
        base = wid * bpw

        pltpu.sync_copy(u_hbm.at[pl.ds(base, bpw)], u_v)
        pltpu.sync_copy(i_hbm.at[pl.ds(base, bpw)], i_v)

        def start_chunk(ch):
            b = ch % NBUF
            cp_p = pltpu.async_copy(
                p_hbm.at[u_v.at[pl.ds(ch * C, C)]], p_buf.at[b], sems[b])
            cp_q = pltpu.async_copy(
                q_hbm.at[i_v.at[pl.ds(ch * C, C)]], q_buf.at[b], sems[b])
            return cp_p, cp_q

        pending = start_chunk(0)
        for ch in range(nch):
            b = ch % NBUF
            pending[0].wait()
            pending[1].wait()
            if ch + 1 < nch:
                pending = start_chunk(ch + 1)
            _dot_chunk(p_buf.at[b], q_buf.at[b], out_v, ch * C)

        pltpu.sync_copy(out_v, out_hbm.at[pl.ds(base, bpw)])

    return run(u_idx, i_idx, P, Q)


# speedup vs baseline: 1.4011x; 1.1864x over previous
import functools

import jax
import jax.numpy as jnp
from jax import lax
from jax.experimental import pallas as pl
from jax.experimental.pallas import tpu as pltpu
from jax.experimental.pallas import tpu_sc as plsc

B = 16384
F = 128
C = 128
NBUF = 2


def _dot_chunk(p_ref, q_ref, tmp_ref, out_ref, out_base):
    lanes = lax.iota(jnp.int32, 16)
    perms = [lanes ^ s for s in (8, 4, 2, 1)]
    lane_eq = [lanes == l for l in range(1, 16)]

    def row(r, carry):
        acc = p_ref[r, pl.ds(0, 16)] * q_ref[r, pl.ds(0, 16)]
        for c in range(1, F // 16):
            acc = acc + p_ref[r, pl.ds(c * 16, 16)] * q_ref[r, pl.ds(c * 16, 16)]
        for perm in perms:
            acc = acc + acc.at[perm].get(mode="promise_in_bounds",
                                         unique_indices=True)
        tmp_ref[r, pl.ds(0, 16)] = acc
        return carry

    lax.fori_loop(0, C, row, 0, unroll=4)

    def compact(g, carry):
        res = tmp_ref[g * 16, pl.ds(0, 16)]
        for l in range(1, 16):
            res = jnp.where(lane_eq[l - 1], tmp_ref[g * 16 + l, pl.ds(0, 16)],
                            res)
        out_ref[pl.ds(out_base + g * 16, 16)] = res
        return carry

    lax.fori_loop(0, C // 16, compact, 0)


def kernel(u_idx, i_idx, P, Q):
    info = plsc.get_sparse_core_info()
    nc, ns = info.num_cores, info.num_subcores
    nw = nc * ns
    bpw = B // nw
    nch = bpw // C

    mesh = plsc.VectorSubcoreMesh(core_axis_name="c", subcore_axis_name="s")

    @functools.partial(
        pl.kernel,
        mesh=mesh,
        out_type=jax.ShapeDtypeStruct((B,), jnp.float32),
        scratch_types=[
            pltpu.VMEM((bpw,), jnp.int32),
            pltpu.VMEM((bpw,), jnp.int32),
            pltpu.VMEM((NBUF, C, F), jnp.float32),
            pltpu.VMEM((NBUF, C, F), jnp.float32),
            pltpu.VMEM((C, 16), jnp.float32),
            pltpu.VMEM((bpw,), jnp.float32),
            pltpu.SemaphoreType.DMA,
            pltpu.SemaphoreType.DMA,
        ],
    )
    def run(u_hbm, i_hbm, p_hbm, q_hbm, out_hbm, u_v, i_v, p_buf, q_buf,
            tmp_v, out_v, sem0, sem1):
        sems = [sem0, sem1]
        wid = lax.axis_index("s") * nc + lax.axis_index("c")
        base = wid * bpw

        pltpu.sync_copy(u_hbm.at[pl.ds(base, bpw)], u_v)
        pltpu.sync_copy(i_hbm.at[pl.ds(base, bpw)], i_v)

        def start_chunk(ch):
            b = ch % NBUF
            cp_p = pltpu.async_copy(
                p_hbm.at[u_v.at[pl.ds(ch * C, C)]], p_buf.at[b], sems[b])
            cp_q = pltpu.async_copy(
                q_hbm.at[i_v.at[pl.ds(ch * C, C)]], q_buf.at[b], sems[b])
            return cp_p, cp_q

        pending = start_chunk(0)
        for ch in range(nch):
            b = ch % NBUF
            pending[0].wait()
            pending[1].wait()
            if ch + 1 < nch:
                pending = start_chunk(ch + 1)
            _dot_chunk(p_buf.at[b], q_buf.at[b], tmp_v, out_v, ch * C)

        pltpu.sync_copy(out_v, out_hbm.at[pl.ds(base, bpw)])

    return run(u_idx, i_idx, P, Q)


# speedup vs baseline: 1.4015x; 1.0003x over previous
import functools

import jax
import jax.numpy as jnp
from jax import lax
from jax.experimental import pallas as pl
from jax.experimental.pallas import tpu as pltpu
from jax.experimental.pallas import tpu_sc as plsc

B = 16384
F = 128
C = 128
NBUF = 3


def _dot_chunk(p_ref, q_ref, tmp_ref, out_ref, out_base):
    lanes = lax.iota(jnp.int32, 16)
    perms = [lanes ^ s for s in (8, 4, 2, 1)]
    lane_eq = [lanes == l for l in range(1, 16)]

    def row(r, carry):
        acc = p_ref[r, pl.ds(0, 16)] * q_ref[r, pl.ds(0, 16)]
        for c in range(1, F // 16):
            acc = acc + p_ref[r, pl.ds(c * 16, 16)] * q_ref[r, pl.ds(c * 16, 16)]
        for perm in perms:
            acc = acc + acc.at[perm].get(mode="promise_in_bounds",
                                         unique_indices=True)
        tmp_ref[r, pl.ds(0, 16)] = acc
        return carry

    lax.fori_loop(0, C, row, 0, unroll=4)

    def compact(g, carry):
        res = tmp_ref[g * 16, pl.ds(0, 16)]
        for l in range(1, 16):
            res = jnp.where(lane_eq[l - 1], tmp_ref[g * 16 + l, pl.ds(0, 16)],
                            res)
        out_ref[pl.ds(out_base + g * 16, 16)] = res
        return carry

    lax.fori_loop(0, C // 16, compact, 0)


def kernel(u_idx, i_idx, P, Q):
    info = plsc.get_sparse_core_info()
    nc, ns = info.num_cores, info.num_subcores
    nw = nc * ns
    bpw = B // nw
    nch = bpw // C

    mesh = plsc.VectorSubcoreMesh(core_axis_name="c", subcore_axis_name="s")

    @functools.partial(
        pl.kernel,
        mesh=mesh,
        out_type=jax.ShapeDtypeStruct((B,), jnp.float32),
        scratch_types=[
            pltpu.VMEM((bpw,), jnp.int32),
            pltpu.VMEM((bpw,), jnp.int32),
            pltpu.VMEM((NBUF, C, F), jnp.float32),
            pltpu.VMEM((NBUF, C, F), jnp.float32),
            pltpu.VMEM((C, 16), jnp.float32),
            pltpu.VMEM((bpw,), jnp.float32),
            pltpu.SemaphoreType.DMA,
            pltpu.SemaphoreType.DMA,
            pltpu.SemaphoreType.DMA,
            pltpu.SemaphoreType.DMA,
        ],
    )
    def run(u_hbm, i_hbm, p_hbm, q_hbm, out_hbm, u_v, i_v, p_buf, q_buf,
            tmp_v, out_v, idx_sem, sem0, sem1, sem2):
        sems = [sem0, sem1, sem2]
        wid = lax.axis_index("s") * nc + lax.axis_index("c")
        base = wid * bpw

        cp_u = pltpu.async_copy(u_hbm.at[pl.ds(base, bpw)], u_v, idx_sem)
        cp_i = pltpu.async_copy(i_hbm.at[pl.ds(base, bpw)], i_v, idx_sem)
        cp_u.wait()
        cp_i.wait()

        def start_chunk(ch):
            b = ch % NBUF
            cp_p = pltpu.async_copy(
                p_hbm.at[u_v.at[pl.ds(ch * C, C)]], p_buf.at[b], sems[b])
            cp_q = pltpu.async_copy(
                q_hbm.at[i_v.at[pl.ds(ch * C, C)]], q_buf.at[b], sems[b])
            return cp_p, cp_q

        pending = [start_chunk(0), start_chunk(1), None]
        for ch in range(nch):
            cp_p, cp_q = pending[ch % NBUF]
            cp_p.wait()
            cp_q.wait()
            if ch + 2 < nch:
                pending[(ch + 2) % NBUF] = start_chunk(ch + 2)
            _dot_chunk(p_buf.at[ch % NBUF], q_buf.at[ch % NBUF], tmp_v, out_v,
                       ch * C)

        pltpu.sync_copy(out_v, out_hbm.at[pl.ds(base, bpw)])

    return run(u_idx, i_idx, P, Q)
